# baseline (speedup 1.0000x reference)
import functools

import jax
import jax.numpy as jnp
from jax import lax
from jax.experimental import pallas as pl
from jax.experimental.pallas import tpu as pltpu
from jax.experimental.pallas import tpu_sc as plsc

N_NODES = 10000
NIN = 128
NOUT = 128
N_EDGES = 320000
NH = NIN // 2

_NC = 2
_NS = 16
_NW = _NC * _NS
_EW = N_EDGES // _NW
_C = 80
_CH = _EW // _C


def _node_body(x_ref, wlt_ref, bl_ref, w1at_ref, y_ref):
    xl = jnp.dot(x_ref[...], wlt_ref[...], preferred_element_type=jnp.float32)
    xl = xl + bl_ref[...]
    y = jnp.dot(xl, w1at_ref[...], preferred_element_type=jnp.float32)
    yb = y.astype(jnp.bfloat16)
    lo16 = lax.bitcast_convert_type(yb[:, :NH], jnp.uint16)
    hi16 = lax.bitcast_convert_type(yb[:, NH:], jnp.uint16)
    packed = (hi16.astype(jnp.uint32) << 16) | lo16.astype(jnp.uint32)
    y_ref[...] = lax.bitcast_convert_type(packed, jnp.int32)


def _node_precompute(x, WlT, bl2, W1aT):
    return pl.pallas_call(
        _node_body,
        out_shape=jax.ShapeDtypeStruct((N_NODES, NH), jnp.int32),
    )(x, WlT, bl2, W1aT)


def _gather_add_body(y_hbm, src_hbm, dst_hbm, g_hbm,
                     sidx, didx, srows, drows, sem_s, sem_d):
    wid = lax.axis_index("s") * _NC + lax.axis_index("c")
    pltpu.sync_copy(src_hbm.at[wid], sidx)
    pltpu.sync_copy(dst_hbm.at[wid], didx)

    def chunk(i, _):
        cs = pltpu.async_copy(y_hbm.at[sidx.at[i]], srows, sem_s)
        cd = pltpu.async_copy(y_hbm.at[didx.at[i]], drows, sem_d)
        cs.wait()
        cd.wait()

        def row(r, _):
            def col(j, _):
                k = j * 16
                a = plsc.bitcast(srows[r, pl.ds(k, 16)], jnp.bfloat16)
                b = plsc.bitcast(drows[r, pl.ds(k, 16)], jnp.bfloat16)
                srows[r, pl.ds(k, 16)] = plsc.bitcast(a + b, jnp.int32)
                return 0
            return lax.fori_loop(0, NH // 16, col, 0)

        lax.fori_loop(0, _C, row, 0)
        base = wid * _EW + i * _C
        pltpu.sync_copy(srows, g_hbm.at[pl.ds(base, _C)])
        return 0

    lax.fori_loop(0, _CH, chunk, 0)


def _gather_add(y, src3, dst3):
    mesh = plsc.VectorSubcoreMesh(core_axis_name="c", subcore_axis_name="s")
    fn = functools.partial(
        pl.kernel, mesh=mesh,
        compiler_params=pltpu.CompilerParams(
            needs_layout_passes=False, use_tc_tiling_on_sc=False),
        out_type=jax.ShapeDtypeStruct((N_EDGES, NH), jnp.int32),
        scratch_types=[
            pltpu.VMEM((_CH, _C), jnp.int32),
            pltpu.VMEM((_CH, _C), jnp.int32),
            pltpu.VMEM((_C, NH), jnp.int32),
            pltpu.VMEM((_C, NH), jnp.int32),
            pltpu.SemaphoreType.DMA,
            pltpu.SemaphoreType.DMA,
        ],
    )(_gather_add_body)
    return fn(y, src3, dst3)


_EB = 2560


def _edge_body(g_ref, ea_ref, w1et_ref, b1_ref, w2tlo_ref, w2thi_ref,
               b2_ref, out_ref):
    gp = g_ref[...]
    glo = lax.bitcast_convert_type(gp << 16, jnp.float32)
    ghi = lax.bitcast_convert_type(gp & jnp.int32(-65536), jnp.float32)
    u = jnp.dot(ea_ref[...], w1et_ref[...],
                preferred_element_type=jnp.float32) + b1_ref[...]
    h_lo = jnp.maximum(u[:, :NH] + glo, 0.0)
    h_hi = jnp.maximum(u[:, NH:] + ghi, 0.0)
    o = (jnp.dot(h_lo, w2tlo_ref[...], preferred_element_type=jnp.float32)
         + jnp.dot(h_hi, w2thi_ref[...], preferred_element_type=jnp.float32)
         + b2_ref[...])
    out_ref[...] = jnp.maximum(o, 0.0)


def _edge_mlp(g, edge_attr, W1eT, b12, W2Tlo, W2Thi, b22):
    nblk = N_EDGES // _EB
    return pl.pallas_call(
        _edge_body,
        grid=(nblk,),
        in_specs=[
            pl.BlockSpec((_EB, NH), lambda i: (i, 0)),
            pl.BlockSpec((_EB, NIN), lambda i: (i, 0)),
            pl.BlockSpec((NIN, NOUT), lambda i: (0, 0)),
            pl.BlockSpec((1, NOUT), lambda i: (0, 0)),
            pl.BlockSpec((NH, NOUT), lambda i: (0, 0)),
            pl.BlockSpec((NH, NOUT), lambda i: (0, 0)),
            pl.BlockSpec((1, NOUT), lambda i: (0, 0)),
        ],
        out_specs=pl.BlockSpec((_EB, NOUT), lambda i: (i, 0)),
        out_shape=jax.ShapeDtypeStruct((N_EDGES, NOUT), jnp.float32),
    )(g, edge_attr, W1eT, b12, W2Tlo, W2Thi, b22)


def kernel(x, edge_index, edge_attr, Wl, bl, W1, b1, W2, b2):
    src3 = edge_index[0].astype(jnp.int32).reshape(_NW, _CH, _C)
    dst3 = edge_index[1].astype(jnp.int32).reshape(_NW, _CH, _C)
    WlT = Wl.T
    W1aT = W1[:, :NIN].T
    W1eT = W1[:, NIN:].T
    W2T = W2.T

    y = _node_precompute(x, WlT, bl.reshape(1, NIN), W1aT)
    g = _gather_add(y, src3, dst3)
    return _edge_mlp(g, edge_attr, W1eT, b1.reshape(1, NOUT),
                     W2T[:NH], W2T[NH:], b2.reshape(1, NOUT))


# speedup vs baseline: 1.3443x; 1.3443x over previous
import functools

import jax
import jax.numpy as jnp
from jax import lax
from jax.experimental import pallas as pl
from jax.experimental.pallas import tpu as pltpu
from jax.experimental.pallas import tpu_sc as plsc

N_NODES = 10000
NIN = 128
NOUT = 128
N_EDGES = 320000
NH = NIN // 2

_NC = 2
_NS = 16
_NW = _NC * _NS
_EW = N_EDGES // _NW
_C = 80
_CH = _EW // _C


def _node_body(x_ref, wlt_ref, bl_ref, w1at_ref, y_ref):
    xl = jnp.dot(x_ref[...], wlt_ref[...], preferred_element_type=jnp.float32)
    xl = xl + bl_ref[...]
    y = jnp.dot(xl, w1at_ref[...], preferred_element_type=jnp.float32)
    yb = y.astype(jnp.bfloat16)
    lo16 = lax.bitcast_convert_type(yb[:, :NH], jnp.uint16)
    hi16 = lax.bitcast_convert_type(yb[:, NH:], jnp.uint16)
    packed = (hi16.astype(jnp.uint32) << 16) | lo16.astype(jnp.uint32)
    y_ref[...] = lax.bitcast_convert_type(packed, jnp.int32)


def _node_precompute(x, WlT, bl2, W1aT):
    return pl.pallas_call(
        _node_body,
        out_shape=jax.ShapeDtypeStruct((N_NODES, NH), jnp.int32),
    )(x, WlT, bl2, W1aT)


_ILV = plsc.PackFormat.INTERLEAVED


def _gather_add_body(y_hbm, src_hbm, dst_hbm, g_hbm,
                     sidx, didx, srows, drows, obuf, sem_s, sem_d):
    wid = lax.axis_index("s") * _NC + lax.axis_index("c")
    pltpu.sync_copy(src_hbm.at[wid], sidx)
    pltpu.sync_copy(dst_hbm.at[wid], didx)

    def chunk(i, _):
        cs = pltpu.async_copy(y_hbm.at[sidx.at[i]], srows, sem_s)
        cd = pltpu.async_copy(y_hbm.at[didx.at[i]], drows, sem_d)
        cs.wait()
        cd.wait()

        def pair(rr, _):
            e0 = rr * 2
            e1 = e0 + 1

            def kchunk(k, _):
                c = k * 16
                s0 = plsc.bitcast(srows[e0, pl.ds(c, 16)], jnp.bfloat16)
                d0 = plsc.bitcast(drows[e0, pl.ds(c, 16)], jnp.bfloat16)
                s1 = plsc.bitcast(srows[e1, pl.ds(c, 16)], jnp.bfloat16)
                d1 = plsc.bitcast(drows[e1, pl.ds(c, 16)], jnp.bfloat16)
                sa0, sb0 = plsc.unpack(s0, format=_ILV)
                da0, db0 = plsc.unpack(d0, format=_ILV)
                sa1, sb1 = plsc.unpack(s1, format=_ILV)
                da1, db1 = plsc.unpack(d1, format=_ILV)
                fa0 = sa0 + da0
                fb0 = sb0 + db0
                fa1 = sa1 + da1
                fb1 = sb1 + db1
                lo_pair = plsc.bitcast(plsc.pack(fa0, fa1, format=_ILV),
                                       jnp.int32)
                hi_pair = plsc.bitcast(plsc.pack(fb0, fb1, format=_ILV),
                                       jnp.int32)
                obuf[rr, pl.ds(c, 16)] = lo_pair
                obuf[rr, pl.ds(NH + c, 16)] = hi_pair
                return 0

            return lax.fori_loop(0, NH // 16, kchunk, 0)

        lax.fori_loop(0, _C // 2, pair, 0)
        base2 = wid * (_EW // 2) + i * (_C // 2)
        pltpu.sync_copy(obuf, g_hbm.at[pl.ds(base2, _C // 2)])
        return 0

    lax.fori_loop(0, _CH, chunk, 0)


def _gather_add(y, src3, dst3):
    mesh = plsc.VectorSubcoreMesh(core_axis_name="c", subcore_axis_name="s")
    fn = functools.partial(
        pl.kernel, mesh=mesh,
        compiler_params=pltpu.CompilerParams(
            needs_layout_passes=False, use_tc_tiling_on_sc=False),
        out_type=jax.ShapeDtypeStruct((N_EDGES // 2, NIN), jnp.int32),
        scratch_types=[
            pltpu.VMEM((_CH, _C), jnp.int32),
            pltpu.VMEM((_CH, _C), jnp.int32),
            pltpu.VMEM((_C, NH), jnp.int32),
            pltpu.VMEM((_C, NH), jnp.int32),
            pltpu.VMEM((_C // 2, NIN), jnp.int32),
            pltpu.SemaphoreType.DMA,
            pltpu.SemaphoreType.DMA,
        ],
    )(_gather_add_body)
    return fn(y, src3, dst3)


_EB = 2560


def _edge_body(g_ref, ea_ref, w1et_ref, b1_ref, w2t_ref, b2_ref, out_ref):
    gbf = pltpu.bitcast(g_ref[...], jnp.bfloat16)
    gf = gbf.astype(jnp.float32)
    u = jnp.dot(ea_ref[...], w1et_ref[...],
                preferred_element_type=jnp.float32) + b1_ref[...]
    h = jnp.maximum(u + gf, 0.0)
    o = jnp.dot(h, w2t_ref[...], preferred_element_type=jnp.float32) + b2_ref[...]
    out_ref[...] = jnp.maximum(o, 0.0)


def _edge_mlp(g, edge_attr, W1eT, b12, W2T, b22):
    nblk = N_EDGES // _EB
    return pl.pallas_call(
        _edge_body,
        grid=(nblk,),
        in_specs=[
            pl.BlockSpec((_EB // 2, NIN), lambda i: (i, 0)),
            pl.BlockSpec((_EB, NIN), lambda i: (i, 0)),
            pl.BlockSpec((NIN, NOUT), lambda i: (0, 0)),
            pl.BlockSpec((1, NOUT), lambda i: (0, 0)),
            pl.BlockSpec((NOUT, NOUT), lambda i: (0, 0)),
            pl.BlockSpec((1, NOUT), lambda i: (0, 0)),
        ],
        out_specs=pl.BlockSpec((_EB, NOUT), lambda i: (i, 0)),
        out_shape=jax.ShapeDtypeStruct((N_EDGES, NOUT), jnp.float32),
    )(g, edge_attr, W1eT, b12, W2T, b22)


def kernel(x, edge_index, edge_attr, Wl, bl, W1, b1, W2, b2):
    src3 = edge_index[0].astype(jnp.int32).reshape(_NW, _CH, _C)
    dst3 = edge_index[1].astype(jnp.int32).reshape(_NW, _CH, _C)
    WlT = Wl.T
    W1aT = W1[:, :NIN].T
    W1eT = W1[:, NIN:].T
    W2T = W2.T

    y = _node_precompute(x, WlT, bl.reshape(1, NIN), W1aT)
    g = _gather_add(y, src3, dst3)
    return _edge_mlp(g, edge_attr, W1eT, b1.reshape(1, NOUT), W2T,
                     b2.reshape(1, NOUT))


# speedup vs baseline: 1.7030x; 1.2669x over previous
import functools

import jax
import jax.numpy as jnp
from jax import lax
from jax.experimental import pallas as pl
from jax.experimental.pallas import tpu as pltpu
from jax.experimental.pallas import tpu_sc as plsc

N_NODES = 10000
NIN = 128
NOUT = 128
N_EDGES = 320000
NH = NIN // 2

_NC = 2
_NS = 16
_NW = _NC * _NS
_K = 5
_EK = N_EDGES // _K
_EW = _EK // _NW
_C = 80
_CH = _EW // _C


def _node_body(x_ref, wlt_ref, bl_ref, w1at_ref, y_ref):
    xl = jnp.dot(x_ref[...], wlt_ref[...], preferred_element_type=jnp.float32)
    xl = xl + bl_ref[...]
    y = jnp.dot(xl, w1at_ref[...], preferred_element_type=jnp.float32)
    yb = y.astype(jnp.bfloat16)
    lo16 = lax.bitcast_convert_type(yb[:, :NH], jnp.uint16)
    hi16 = lax.bitcast_convert_type(yb[:, NH:], jnp.uint16)
    packed = (hi16.astype(jnp.uint32) << 16) | lo16.astype(jnp.uint32)
    y_ref[...] = lax.bitcast_convert_type(packed, jnp.int32)


def _node_precompute(x, WlT, bl2, W1aT):
    return pl.pallas_call(
        _node_body,
        out_shape=jax.ShapeDtypeStruct((N_NODES, NH), jnp.int32),
    )(x, WlT, bl2, W1aT)


_ILV = plsc.PackFormat.INTERLEAVED


def _gather_add_body(y_hbm, src_hbm, dst_hbm, g_hbm,
                     sidx, didx, srows, drows, obuf, sem_s, sem_d):
    wid = lax.axis_index("s") * _NC + lax.axis_index("c")
    pltpu.sync_copy(src_hbm.at[wid], sidx)
    pltpu.sync_copy(dst_hbm.at[wid], didx)

    def chunk(i, _):
        cs = pltpu.async_copy(y_hbm.at[sidx.at[i]], srows, sem_s)
        cd = pltpu.async_copy(y_hbm.at[didx.at[i]], drows, sem_d)
        cs.wait()
        cd.wait()

        def pair(rr, _):
            e0 = rr * 2
            e1 = e0 + 1

            def kchunk(k, _):
                c = k * 16
                s0 = plsc.bitcast(srows[e0, pl.ds(c, 16)], jnp.bfloat16)
                d0 = plsc.bitcast(drows[e0, pl.ds(c, 16)], jnp.bfloat16)
                s1 = plsc.bitcast(srows[e1, pl.ds(c, 16)], jnp.bfloat16)
                d1 = plsc.bitcast(drows[e1, pl.ds(c, 16)], jnp.bfloat16)
                sa0, sb0 = plsc.unpack(s0, format=_ILV)
                da0, db0 = plsc.unpack(d0, format=_ILV)
                sa1, sb1 = plsc.unpack(s1, format=_ILV)
                da1, db1 = plsc.unpack(d1, format=_ILV)
                fa0 = sa0 + da0
                fb0 = sb0 + db0
                fa1 = sa1 + da1
                fb1 = sb1 + db1
                lo_pair = plsc.bitcast(plsc.pack(fa0, fa1, format=_ILV),
                                       jnp.int32)
                hi_pair = plsc.bitcast(plsc.pack(fb0, fb1, format=_ILV),
                                       jnp.int32)
                obuf[rr, pl.ds(c, 16)] = lo_pair
                obuf[rr, pl.ds(NH + c, 16)] = hi_pair
                return 0

            return lax.fori_loop(0, NH // 16, kchunk, 0)

        lax.fori_loop(0, _C // 2, pair, 0)
        base2 = wid * (_EW // 2) + i * (_C // 2)
        pltpu.sync_copy(obuf, g_hbm.at[pl.ds(base2, _C // 2)])
        return 0

    lax.fori_loop(0, _CH, chunk, 0)


def _gather_add(y, src3, dst3):
    mesh = plsc.VectorSubcoreMesh(core_axis_name="c", subcore_axis_name="s")
    fn = functools.partial(
        pl.kernel, mesh=mesh,
        compiler_params=pltpu.CompilerParams(
            needs_layout_passes=False, use_tc_tiling_on_sc=False),
        out_type=jax.ShapeDtypeStruct((_EK // 2, NIN), jnp.int32),
        scratch_types=[
            pltpu.VMEM((_CH, _C), jnp.int32),
            pltpu.VMEM((_CH, _C), jnp.int32),
            pltpu.VMEM((_C, NH), jnp.int32),
            pltpu.VMEM((_C, NH), jnp.int32),
            pltpu.VMEM((_C // 2, NIN), jnp.int32),
            pltpu.SemaphoreType.DMA,
            pltpu.SemaphoreType.DMA,
        ],
    )(_gather_add_body)
    return fn(y, src3, dst3)


_EB = 2560
_NBK = _EK // _EB


def _edge_body(g_ref, ea_ref, w1et_ref, b1_ref, w2t_ref, b2_ref, out_ref):
    gbf = pltpu.bitcast(g_ref[...], jnp.bfloat16)
    gf = gbf.astype(jnp.float32)
    u = jnp.dot(ea_ref[...], w1et_ref[...],
                preferred_element_type=jnp.float32) + b1_ref[...]
    h = jnp.maximum(u + gf, 0.0)
    o = jnp.dot(h, w2t_ref[...], preferred_element_type=jnp.float32) + b2_ref[...]
    out_ref[...] = jnp.maximum(o, 0.0)


def _edge_body_prev(g_ref, ea_ref, w1et_ref, b1_ref, w2t_ref, b2_ref,
                    prev_ref, out_ref):
    del prev_ref
    _edge_body(g_ref, ea_ref, w1et_ref, b1_ref, w2t_ref, b2_ref, out_ref)


def _edge_mlp_slice(k, g_k, edge_attr, W1eT, b12, W2T, b22, out_prev):
    off = k * _NBK
    in_specs = [
        pl.BlockSpec((_EB // 2, NIN), lambda i: (i, 0)),
        pl.BlockSpec((_EB, NIN), lambda i, o=off: (i + o, 0)),
        pl.BlockSpec((NIN, NOUT), lambda i: (0, 0)),
        pl.BlockSpec((1, NOUT), lambda i: (0, 0)),
        pl.BlockSpec((NOUT, NOUT), lambda i: (0, 0)),
        pl.BlockSpec((1, NOUT), lambda i: (0, 0)),
    ]
    args = [g_k, edge_attr, W1eT, b12, W2T, b22]
    body = _edge_body
    aliases = {}
    if out_prev is not None:
        in_specs.append(pl.BlockSpec(memory_space=pl.MemorySpace.ANY))
        args.append(out_prev)
        body = _edge_body_prev
        aliases = {6: 0}
    return pl.pallas_call(
        body,
        grid=(_NBK,),
        in_specs=in_specs,
        out_specs=pl.BlockSpec((_EB, NOUT), lambda i, o=off: (i + o, 0)),
        out_shape=jax.ShapeDtypeStruct((N_EDGES, NOUT), jnp.float32),
        input_output_aliases=aliases,
    )(*args)


def kernel(x, edge_index, edge_attr, Wl, bl, W1, b1, W2, b2):
    src4 = edge_index[0].astype(jnp.int32).reshape(_K, _NW, _CH, _C)
    dst4 = edge_index[1].astype(jnp.int32).reshape(_K, _NW, _CH, _C)
    WlT = Wl.T
    W1aT = W1[:, :NIN].T
    W1eT = W1[:, NIN:].T
    W2T = W2.T
    b12 = b1.reshape(1, NOUT)
    b22 = b2.reshape(1, NOUT)

    y = _node_precompute(x, WlT, bl.reshape(1, NIN), W1aT)
    gs = [_gather_add(y, src4[k], dst4[k]) for k in range(_K)]
    out = None
    for k in range(_K):
        out = _edge_mlp_slice(k, gs[k], edge_attr, W1eT, b12, W2T, b22, out)
    return out


# speedup vs baseline: 2.0275x; 1.1905x over previous
import functools

import jax
import jax.numpy as jnp
from jax import lax
from jax.experimental import pallas as pl
from jax.experimental.pallas import tpu as pltpu
from jax.experimental.pallas import tpu_sc as plsc

N_NODES = 10000
NIN = 128
NOUT = 128
N_EDGES = 320000
NH = NIN // 2

_NC = 2
_NS = 16
_NW = _NC * _NS
_K = 5
_EK = N_EDGES // _K
_EW = _EK // _NW
_C = 80
_CH = _EW // _C


def _node_body(x_ref, wlt_ref, bl_ref, w1at_ref, y_ref):
    xl = jnp.dot(x_ref[...], wlt_ref[...], preferred_element_type=jnp.float32)
    xl = xl + bl_ref[...]
    y = jnp.dot(xl, w1at_ref[...], preferred_element_type=jnp.float32)
    yb = y.astype(jnp.bfloat16)
    lo16 = lax.bitcast_convert_type(yb[:, :NH], jnp.uint16)
    hi16 = lax.bitcast_convert_type(yb[:, NH:], jnp.uint16)
    packed = (hi16.astype(jnp.uint32) << 16) | lo16.astype(jnp.uint32)
    y_ref[...] = lax.bitcast_convert_type(packed, jnp.int32)


def _node_precompute(x, WlT, bl2, W1aT):
    return pl.pallas_call(
        _node_body,
        out_shape=jax.ShapeDtypeStruct((N_NODES, NH), jnp.int32),
    )(x, WlT, bl2, W1aT)


_ILV = plsc.PackFormat.INTERLEAVED


def _gather_add_body(y_hbm, src_hbm, dst_hbm, g_hbm,
                     sidx, didx, sbuf0, dbuf0, sbuf1, dbuf1, obuf,
                     sem_s0, sem_d0, sem_s1, sem_d1):
    wid = lax.axis_index("s") * _NC + lax.axis_index("c")
    pltpu.sync_copy(src_hbm.at[wid], sidx)
    pltpu.sync_copy(dst_hbm.at[wid], didx)

    def issue(i, sbuf, dbuf, ss, sd):
        pltpu.make_async_copy(y_hbm.at[sidx.at[i]], sbuf, ss).start()
        pltpu.make_async_copy(y_hbm.at[didx.at[i]], dbuf, sd).start()

    def drain(sbuf, dbuf, ss, sd):
        pltpu.make_async_copy(y_hbm.at[sidx.at[0]], sbuf, ss).wait()
        pltpu.make_async_copy(y_hbm.at[didx.at[0]], dbuf, sd).wait()

    def compute_store(i, srows, drows):
        def pair(rr, _):
            e0 = rr * 2
            e1 = e0 + 1

            def kchunk(k, _):
                c = k * 16
                s0 = plsc.bitcast(srows[e0, pl.ds(c, 16)], jnp.bfloat16)
                d0 = plsc.bitcast(drows[e0, pl.ds(c, 16)], jnp.bfloat16)
                s1 = plsc.bitcast(srows[e1, pl.ds(c, 16)], jnp.bfloat16)
                d1 = plsc.bitcast(drows[e1, pl.ds(c, 16)], jnp.bfloat16)
                sa0, sb0 = plsc.unpack(s0, format=_ILV)
                da0, db0 = plsc.unpack(d0, format=_ILV)
                sa1, sb1 = plsc.unpack(s1, format=_ILV)
                da1, db1 = plsc.unpack(d1, format=_ILV)
                fa0 = sa0 + da0
                fb0 = sb0 + db0
                fa1 = sa1 + da1
                fb1 = sb1 + db1
                lo_pair = plsc.bitcast(plsc.pack(fa0, fa1, format=_ILV),
                                       jnp.int32)
                hi_pair = plsc.bitcast(plsc.pack(fb0, fb1, format=_ILV),
                                       jnp.int32)
                obuf[rr, pl.ds(c, 16)] = lo_pair
                obuf[rr, pl.ds(NH + c, 16)] = hi_pair
                return 0

            return lax.fori_loop(0, NH // 16, kchunk, 0)

        lax.fori_loop(0, _C // 2, pair, 0)
        base2 = wid * (_EW // 2) + i * (_C // 2)
        pltpu.sync_copy(obuf, g_hbm.at[pl.ds(base2, _C // 2)])

    issue(0, sbuf0, dbuf0, sem_s0, sem_d0)

    def outer(i, _):
        nxt = jnp.minimum(i + 1, _CH - 1)

        @pl.when(i % 2 == 0)
        def _():
            issue(nxt, sbuf1, dbuf1, sem_s1, sem_d1)
            drain(sbuf0, dbuf0, sem_s0, sem_d0)
            compute_store(i, sbuf0, dbuf0)

        @pl.when(i % 2 == 1)
        def _():
            issue(nxt, sbuf0, dbuf0, sem_s0, sem_d0)
            drain(sbuf1, dbuf1, sem_s1, sem_d1)
            compute_store(i, sbuf1, dbuf1)

        return 0

    lax.fori_loop(0, _CH, outer, 0)
    drain(sbuf1, dbuf1, sem_s1, sem_d1)


def _gather_add(y, src3, dst3):
    mesh = plsc.VectorSubcoreMesh(core_axis_name="c", subcore_axis_name="s")
    fn = functools.partial(
        pl.kernel, mesh=mesh,
        compiler_params=pltpu.CompilerParams(
            needs_layout_passes=False, use_tc_tiling_on_sc=False),
        out_type=jax.ShapeDtypeStruct((_EK // 2, NIN), jnp.int32),
        scratch_types=[
            pltpu.VMEM((_CH, _C), jnp.int32),
            pltpu.VMEM((_CH, _C), jnp.int32),
            pltpu.VMEM((_C, NH), jnp.int32),
            pltpu.VMEM((_C, NH), jnp.int32),
            pltpu.VMEM((_C, NH), jnp.int32),
            pltpu.VMEM((_C, NH), jnp.int32),
            pltpu.VMEM((_C // 2, NIN), jnp.int32),
            pltpu.SemaphoreType.DMA,
            pltpu.SemaphoreType.DMA,
            pltpu.SemaphoreType.DMA,
            pltpu.SemaphoreType.DMA,
        ],
    )(_gather_add_body)
    return fn(y, src3, dst3)


_EB = 2560
_NBK = _EK // _EB


def _edge_body(g_ref, ea_ref, w1et_ref, b1_ref, w2t_ref, b2_ref, out_ref):
    gbf = pltpu.bitcast(g_ref[...], jnp.bfloat16)
    gf = gbf.astype(jnp.float32)
    u = jnp.dot(ea_ref[...], w1et_ref[...],
                preferred_element_type=jnp.float32) + b1_ref[...]
    h = jnp.maximum(u + gf, 0.0)
    o = jnp.dot(h, w2t_ref[...], preferred_element_type=jnp.float32) + b2_ref[...]
    out_ref[...] = jnp.maximum(o, 0.0)


def _edge_body_prev(g_ref, ea_ref, w1et_ref, b1_ref, w2t_ref, b2_ref,
                    prev_ref, out_ref):
    del prev_ref
    _edge_body(g_ref, ea_ref, w1et_ref, b1_ref, w2t_ref, b2_ref, out_ref)


def _edge_mlp_slice(k, g_k, edge_attr, W1eT, b12, W2T, b22, out_prev):
    off = k * _NBK
    in_specs = [
        pl.BlockSpec((_EB // 2, NIN), lambda i: (i, 0)),
        pl.BlockSpec((_EB, NIN), lambda i, o=off: (i + o, 0)),
        pl.BlockSpec((NIN, NOUT), lambda i: (0, 0)),
        pl.BlockSpec((1, NOUT), lambda i: (0, 0)),
        pl.BlockSpec((NOUT, NOUT), lambda i: (0, 0)),
        pl.BlockSpec((1, NOUT), lambda i: (0, 0)),
    ]
    args = [g_k, edge_attr, W1eT, b12, W2T, b22]
    body = _edge_body
    aliases = {}
    if out_prev is not None:
        in_specs.append(pl.BlockSpec(memory_space=pl.MemorySpace.ANY))
        args.append(out_prev)
        body = _edge_body_prev
        aliases = {6: 0}
    return pl.pallas_call(
        body,
        grid=(_NBK,),
        in_specs=in_specs,
        out_specs=pl.BlockSpec((_EB, NOUT), lambda i, o=off: (i + o, 0)),
        out_shape=jax.ShapeDtypeStruct((N_EDGES, NOUT), jnp.float32),
        input_output_aliases=aliases,
    )(*args)


def kernel(x, edge_index, edge_attr, Wl, bl, W1, b1, W2, b2):
    src4 = edge_index[0].astype(jnp.int32).reshape(_K, _NW, _CH, _C)
    dst4 = edge_index[1].astype(jnp.int32).reshape(_K, _NW, _CH, _C)
    WlT = Wl.T
    W1aT = W1[:, :NIN].T
    W1eT = W1[:, NIN:].T
    W2T = W2.T
    b12 = b1.reshape(1, NOUT)
    b22 = b2.reshape(1, NOUT)

    y = _node_precompute(x, WlT, bl.reshape(1, NIN), W1aT)
    gs = [_gather_add(y, src4[k], dst4[k]) for k in range(_K)]
    out = None
    for k in range(_K):
        out = _edge_mlp_slice(k, gs[k], edge_attr, W1eT, b12, W2T, b22, out)
    return out


# speedup vs baseline: 2.2296x; 1.0997x over previous
import functools

import jax
import jax.numpy as jnp
from jax import lax
from jax.experimental import pallas as pl
from jax.experimental.pallas import tpu as pltpu
from jax.experimental.pallas import tpu_sc as plsc

N_NODES = 10000
NIN = 128
NOUT = 128
N_EDGES = 320000
NH = NIN // 2

_NC = 2
_NS = 16
_NW = _NC * _NS
_K = 5
_EK = N_EDGES // _K
_EW = _EK // _NW
_C = 80
_CH = _EW // _C


def _node_body(x_ref, wlt_ref, bl_ref, w1at_ref, y_ref):
    xl = jnp.dot(x_ref[...], wlt_ref[...], preferred_element_type=jnp.float32)
    xl = xl + bl_ref[...]
    y = jnp.dot(xl, w1at_ref[...], preferred_element_type=jnp.float32)
    yb = y.astype(jnp.bfloat16)
    lo16 = lax.bitcast_convert_type(yb[:, :NH], jnp.uint16)
    hi16 = lax.bitcast_convert_type(yb[:, NH:], jnp.uint16)
    packed = (hi16.astype(jnp.uint32) << 16) | lo16.astype(jnp.uint32)
    y_ref[...] = lax.bitcast_convert_type(packed, jnp.int32)


def _node_precompute(x, WlT, bl2, W1aT):
    return pl.pallas_call(
        _node_body,
        out_shape=jax.ShapeDtypeStruct((N_NODES, NH), jnp.int32),
    )(x, WlT, bl2, W1aT)


_ILV = plsc.PackFormat.INTERLEAVED


def _gather_add_body(y_hbm, src_hbm, dst_hbm, g_hbm,
                     ytab, sidx, didx, sbuf0, dbuf0, sbuf1, dbuf1, obuf,
                     sem_s0, sem_d0, sem_s1, sem_d1):
    cid = lax.axis_index("c")
    sid = lax.axis_index("s")
    wid = sid * _NC + cid

    @pl.when(sid == 0)
    def _():
        pltpu.sync_copy(y_hbm, ytab)

    pltpu.sync_copy(src_hbm.at[wid], sidx)
    pltpu.sync_copy(dst_hbm.at[wid], didx)
    plsc.subcore_barrier()

    def issue(i, sbuf, dbuf, ss, sd):
        pltpu.make_async_copy(ytab.at[sidx.at[i]], sbuf, ss).start()
        pltpu.make_async_copy(ytab.at[didx.at[i]], dbuf, sd).start()

    def drain(sbuf, dbuf, ss, sd):
        pltpu.make_async_copy(y_hbm.at[sidx.at[0]], sbuf, ss).wait()
        pltpu.make_async_copy(y_hbm.at[didx.at[0]], dbuf, sd).wait()

    def compute_store(i, srows, drows):
        def pair(rr, _):
            e0 = rr * 2
            e1 = e0 + 1

            def kchunk(k, _):
                c = k * 16
                s0 = plsc.bitcast(srows[e0, pl.ds(c, 16)], jnp.bfloat16)
                d0 = plsc.bitcast(drows[e0, pl.ds(c, 16)], jnp.bfloat16)
                s1 = plsc.bitcast(srows[e1, pl.ds(c, 16)], jnp.bfloat16)
                d1 = plsc.bitcast(drows[e1, pl.ds(c, 16)], jnp.bfloat16)
                sa0, sb0 = plsc.unpack(s0, format=_ILV)
                da0, db0 = plsc.unpack(d0, format=_ILV)
                sa1, sb1 = plsc.unpack(s1, format=_ILV)
                da1, db1 = plsc.unpack(d1, format=_ILV)
                fa0 = sa0 + da0
                fb0 = sb0 + db0
                fa1 = sa1 + da1
                fb1 = sb1 + db1
                lo_pair = plsc.bitcast(plsc.pack(fa0, fa1, format=_ILV),
                                       jnp.int32)
                hi_pair = plsc.bitcast(plsc.pack(fb0, fb1, format=_ILV),
                                       jnp.int32)
                obuf[rr, pl.ds(c, 16)] = lo_pair
                obuf[rr, pl.ds(NH + c, 16)] = hi_pair
                return 0

            return lax.fori_loop(0, NH // 16, kchunk, 0)

        lax.fori_loop(0, _C // 2, pair, 0)
        base2 = wid * (_EW // 2) + i * (_C // 2)
        pltpu.sync_copy(obuf, g_hbm.at[pl.ds(base2, _C // 2)])

    issue(0, sbuf0, dbuf0, sem_s0, sem_d0)

    def outer(i, _):
        nxt = jnp.minimum(i + 1, _CH - 1)

        @pl.when(i % 2 == 0)
        def _():
            issue(nxt, sbuf1, dbuf1, sem_s1, sem_d1)
            drain(sbuf0, dbuf0, sem_s0, sem_d0)
            compute_store(i, sbuf0, dbuf0)

        @pl.when(i % 2 == 1)
        def _():
            issue(nxt, sbuf0, dbuf0, sem_s0, sem_d0)
            drain(sbuf1, dbuf1, sem_s1, sem_d1)
            compute_store(i, sbuf1, dbuf1)

        return 0

    lax.fori_loop(0, _CH, outer, 0)
    drain(sbuf1, dbuf1, sem_s1, sem_d1)


def _gather_add(y, src3, dst3):
    mesh = plsc.VectorSubcoreMesh(core_axis_name="c", subcore_axis_name="s")
    fn = functools.partial(
        pl.kernel, mesh=mesh,
        compiler_params=pltpu.CompilerParams(
            needs_layout_passes=False, use_tc_tiling_on_sc=False),
        out_type=jax.ShapeDtypeStruct((_EK // 2, NIN), jnp.int32),
        scratch_types=[
            pltpu.VMEM_SHARED((N_NODES, NH), jnp.int32),
            pltpu.VMEM((_CH, _C), jnp.int32),
            pltpu.VMEM((_CH, _C), jnp.int32),
            pltpu.VMEM((_C, NH), jnp.int32),
            pltpu.VMEM((_C, NH), jnp.int32),
            pltpu.VMEM((_C, NH), jnp.int32),
            pltpu.VMEM((_C, NH), jnp.int32),
            pltpu.VMEM((_C // 2, NIN), jnp.int32),
            pltpu.SemaphoreType.DMA,
            pltpu.SemaphoreType.DMA,
            pltpu.SemaphoreType.DMA,
            pltpu.SemaphoreType.DMA,
        ],
    )(_gather_add_body)
    return fn(y, src3, dst3)


_EB = 2560
_NBK = _EK // _EB


def _edge_body(g_ref, ea_ref, w1et_ref, b1_ref, w2t_ref, b2_ref, out_ref):
    gbf = pltpu.bitcast(g_ref[...], jnp.bfloat16)
    gf = gbf.astype(jnp.float32)
    u = jnp.dot(ea_ref[...], w1et_ref[...],
                preferred_element_type=jnp.float32) + b1_ref[...]
    h = jnp.maximum(u + gf, 0.0)
    o = jnp.dot(h, w2t_ref[...], preferred_element_type=jnp.float32) + b2_ref[...]
    out_ref[...] = jnp.maximum(o, 0.0)


def _edge_body_prev(g_ref, ea_ref, w1et_ref, b1_ref, w2t_ref, b2_ref,
                    prev_ref, out_ref):
    del prev_ref
    _edge_body(g_ref, ea_ref, w1et_ref, b1_ref, w2t_ref, b2_ref, out_ref)


def _edge_mlp_slice(k, g_k, edge_attr, W1eT, b12, W2T, b22, out_prev):
    off = k * _NBK
    in_specs = [
        pl.BlockSpec((_EB // 2, NIN), lambda i: (i, 0)),
        pl.BlockSpec((_EB, NIN), lambda i, o=off: (i + o, 0)),
        pl.BlockSpec((NIN, NOUT), lambda i: (0, 0)),
        pl.BlockSpec((1, NOUT), lambda i: (0, 0)),
        pl.BlockSpec((NOUT, NOUT), lambda i: (0, 0)),
        pl.BlockSpec((1, NOUT), lambda i: (0, 0)),
    ]
    args = [g_k, edge_attr, W1eT, b12, W2T, b22]
    body = _edge_body
    aliases = {}
    if out_prev is not None:
        in_specs.append(pl.BlockSpec(memory_space=pl.MemorySpace.ANY))
        args.append(out_prev)
        body = _edge_body_prev
        aliases = {6: 0}
    return pl.pallas_call(
        body,
        grid=(_NBK,),
        in_specs=in_specs,
        out_specs=pl.BlockSpec((_EB, NOUT), lambda i, o=off: (i + o, 0)),
        out_shape=jax.ShapeDtypeStruct((N_EDGES, NOUT), jnp.float32),
        input_output_aliases=aliases,
    )(*args)


def kernel(x, edge_index, edge_attr, Wl, bl, W1, b1, W2, b2):
    src4 = edge_index[0].astype(jnp.int32).reshape(_K, _NW, _CH, _C)
    dst4 = edge_index[1].astype(jnp.int32).reshape(_K, _NW, _CH, _C)
    WlT = Wl.T
    W1aT = W1[:, :NIN].T
    W1eT = W1[:, NIN:].T
    W2T = W2.T
    b12 = b1.reshape(1, NOUT)
    b22 = b2.reshape(1, NOUT)

    y = _node_precompute(x, WlT, bl.reshape(1, NIN), W1aT)
    gs = [_gather_add(y, src4[k], dst4[k]) for k in range(_K)]
    out = None
    for k in range(_K):
        out = _edge_mlp_slice(k, gs[k], edge_attr, W1eT, b12, W2T, b22, out)
    return out


# speedup vs baseline: 2.5757x; 1.1552x over previous
import functools

import jax
import jax.numpy as jnp
from jax import lax
from jax.experimental import pallas as pl
from jax.experimental.pallas import tpu as pltpu
from jax.experimental.pallas import tpu_sc as plsc

N_NODES = 10000
NIN = 128
NOUT = 128
N_EDGES = 320000
NH = NIN // 2

_NC = 2
_NS = 16
_NW = _NC * _NS
_K = 5
_EK = N_EDGES // _K
_EW = _EK // _NW
_C = 80
_CH = _EW // _C


def _node_body(x_ref, wlt_ref, bl_ref, w1at_ref, y_ref):
    xl = jnp.dot(x_ref[...], wlt_ref[...], preferred_element_type=jnp.float32)
    xl = xl + bl_ref[...]
    y = jnp.dot(xl, w1at_ref[...], preferred_element_type=jnp.float32)
    yb = y.astype(jnp.bfloat16)
    lo16 = lax.bitcast_convert_type(yb[:, :NH], jnp.uint16)
    hi16 = lax.bitcast_convert_type(yb[:, NH:], jnp.uint16)
    packed = (hi16.astype(jnp.uint32) << 16) | lo16.astype(jnp.uint32)
    y_ref[...] = lax.bitcast_convert_type(packed, jnp.int32)


def _node_precompute(x, WlT, bl2, W1aT):
    return pl.pallas_call(
        _node_body,
        out_shape=jax.ShapeDtypeStruct((N_NODES, NH), jnp.int32),
    )(x, WlT, bl2, W1aT)


_ILV = plsc.PackFormat.INTERLEAVED


def _gather_add_body(y_hbm, src_hbm, dst_hbm, g_hbm,
                     ytab, sidx, didx, sbuf0, dbuf0, sbuf1, dbuf1, obuf,
                     sem_s0, sem_d0, sem_s1, sem_d1):
    cid = lax.axis_index("c")
    sid = lax.axis_index("s")
    wid = sid * _NC + cid

    @pl.when(sid == 0)
    def _():
        pltpu.sync_copy(y_hbm, ytab)

    pltpu.sync_copy(src_hbm.at[wid], sidx)
    pltpu.sync_copy(dst_hbm.at[wid], didx)
    plsc.subcore_barrier()

    def issue(i, sbuf, dbuf, ss, sd):
        pltpu.make_async_copy(ytab.at[sidx.at[i]], sbuf, ss).start()
        pltpu.make_async_copy(ytab.at[didx.at[i]], dbuf, sd).start()

    def drain(sbuf, dbuf, ss, sd):
        pltpu.make_async_copy(y_hbm.at[sidx.at[0]], sbuf, ss).wait()
        pltpu.make_async_copy(y_hbm.at[didx.at[0]], dbuf, sd).wait()

    def compute_store(i, srows, drows):
        def pair(rr, _):
            e0 = rr * 2
            e1 = e0 + 1

            def kchunk(k, _):
                c = k * 16
                s0 = plsc.bitcast(srows[e0, pl.ds(c, 16)], jnp.bfloat16)
                d0 = plsc.bitcast(drows[e0, pl.ds(c, 16)], jnp.bfloat16)
                s1 = plsc.bitcast(srows[e1, pl.ds(c, 16)], jnp.bfloat16)
                d1 = plsc.bitcast(drows[e1, pl.ds(c, 16)], jnp.bfloat16)
                sa0, sb0 = plsc.unpack(s0, format=_ILV)
                da0, db0 = plsc.unpack(d0, format=_ILV)
                sa1, sb1 = plsc.unpack(s1, format=_ILV)
                da1, db1 = plsc.unpack(d1, format=_ILV)
                fa0 = sa0 + da0
                fb0 = sb0 + db0
                fa1 = sa1 + da1
                fb1 = sb1 + db1
                lo_pair = plsc.bitcast(plsc.pack(fa0, fa1, format=_ILV),
                                       jnp.int32)
                hi_pair = plsc.bitcast(plsc.pack(fb0, fb1, format=_ILV),
                                       jnp.int32)
                obuf[rr, pl.ds(c, 16)] = lo_pair
                obuf[rr, pl.ds(NH + c, 16)] = hi_pair
                return 0

            return lax.fori_loop(0, NH // 16, kchunk, 0)

        lax.fori_loop(0, _C // 2, pair, 0)
        base2 = wid * (_EW // 2) + i * (_C // 2)
        pltpu.sync_copy(obuf, g_hbm.at[pl.ds(base2, _C // 2)])

    issue(0, sbuf0, dbuf0, sem_s0, sem_d0)

    def outer(i, _):
        nxt = jnp.minimum(i + 1, _CH - 1)

        @pl.when(i % 2 == 0)
        def _():
            issue(nxt, sbuf1, dbuf1, sem_s1, sem_d1)
            drain(sbuf0, dbuf0, sem_s0, sem_d0)
            compute_store(i, sbuf0, dbuf0)

        @pl.when(i % 2 == 1)
        def _():
            issue(nxt, sbuf0, dbuf0, sem_s0, sem_d0)
            drain(sbuf1, dbuf1, sem_s1, sem_d1)
            compute_store(i, sbuf1, dbuf1)

        return 0

    lax.fori_loop(0, _CH, outer, 0)
    drain(sbuf1, dbuf1, sem_s1, sem_d1)


def _gather_add(y, src3, dst3):
    mesh = plsc.VectorSubcoreMesh(core_axis_name="c", subcore_axis_name="s")
    fn = functools.partial(
        pl.kernel, mesh=mesh,
        compiler_params=pltpu.CompilerParams(
            needs_layout_passes=False, use_tc_tiling_on_sc=False),
        out_type=jax.ShapeDtypeStruct((_EK // 2, NIN), jnp.int32),
        scratch_types=[
            pltpu.VMEM_SHARED((N_NODES, NH), jnp.int32),
            pltpu.VMEM((_CH, _C), jnp.int32),
            pltpu.VMEM((_CH, _C), jnp.int32),
            pltpu.VMEM((_C, NH), jnp.int32),
            pltpu.VMEM((_C, NH), jnp.int32),
            pltpu.VMEM((_C, NH), jnp.int32),
            pltpu.VMEM((_C, NH), jnp.int32),
            pltpu.VMEM((_C // 2, NIN), jnp.int32),
            pltpu.SemaphoreType.DMA,
            pltpu.SemaphoreType.DMA,
            pltpu.SemaphoreType.DMA,
            pltpu.SemaphoreType.DMA,
        ],
    )(_gather_add_body)
    return fn(y, src3, dst3)


_EB = 6400
_NBK = _EK // _EB


def _edge_body(g_ref, ea_ref, w1et_ref, b1_ref, w2t_ref, b2_ref, out_ref):
    gbf = pltpu.bitcast(g_ref[...], jnp.bfloat16)
    gf = gbf.astype(jnp.float32)
    u = jnp.dot(ea_ref[...], w1et_ref[...],
                preferred_element_type=jnp.float32) + b1_ref[...]
    h = jnp.maximum(u + gf, 0.0)
    o = jnp.dot(h, w2t_ref[...], preferred_element_type=jnp.float32) + b2_ref[...]
    out_ref[...] = jnp.maximum(o, 0.0)


def _edge_body_prev(g_ref, ea_ref, w1et_ref, b1_ref, w2t_ref, b2_ref,
                    prev_ref, out_ref):
    del prev_ref
    _edge_body(g_ref, ea_ref, w1et_ref, b1_ref, w2t_ref, b2_ref, out_ref)


def _edge_mlp_slice(k, g_k, edge_attr, W1eT, b12, W2T, b22, out_prev):
    off = k * _NBK
    in_specs = [
        pl.BlockSpec((_EB // 2, NIN), lambda i: (i, 0)),
        pl.BlockSpec((_EB, NIN), lambda i, o=off: (i + o, 0)),
        pl.BlockSpec((NIN, NOUT), lambda i: (0, 0)),
        pl.BlockSpec((1, NOUT), lambda i: (0, 0)),
        pl.BlockSpec((NOUT, NOUT), lambda i: (0, 0)),
        pl.BlockSpec((1, NOUT), lambda i: (0, 0)),
    ]
    args = [g_k, edge_attr, W1eT, b12, W2T, b22]
    body = _edge_body
    aliases = {}
    if out_prev is not None:
        in_specs.append(pl.BlockSpec(memory_space=pl.MemorySpace.ANY))
        args.append(out_prev)
        body = _edge_body_prev
        aliases = {6: 0}
    return pl.pallas_call(
        body,
        grid=(_NBK,),
        in_specs=in_specs,
        out_specs=pl.BlockSpec((_EB, NOUT), lambda i, o=off: (i + o, 0)),
        out_shape=jax.ShapeDtypeStruct((N_EDGES, NOUT), jnp.float32),
        input_output_aliases=aliases,
    )(*args)


def kernel(x, edge_index, edge_attr, Wl, bl, W1, b1, W2, b2):
    src4 = edge_index[0].astype(jnp.int32).reshape(_K, _NW, _CH, _C)
    dst4 = edge_index[1].astype(jnp.int32).reshape(_K, _NW, _CH, _C)
    WlT = Wl.T
    W1aT = W1[:, :NIN].T
    W1eT = W1[:, NIN:].T
    W2T = W2.T
    b12 = b1.reshape(1, NOUT)
    b22 = b2.reshape(1, NOUT)

    y = _node_precompute(x, WlT, bl.reshape(1, NIN), W1aT)
    gs = [_gather_add(y, src4[k], dst4[k]) for k in range(_K)]
    out = None
    for k in range(_K):
        out = _edge_mlp_slice(k, gs[k], edge_attr, W1eT, b12, W2T, b22, out)
    return out


# speedup vs baseline: 2.6442x; 1.0266x over previous
import functools

import jax
import jax.numpy as jnp
from jax import lax
from jax.experimental import pallas as pl
from jax.experimental.pallas import tpu as pltpu
from jax.experimental.pallas import tpu_sc as plsc

N_NODES = 10000
NIN = 128
NOUT = 128
N_EDGES = 320000
NH = NIN // 2

_NC = 2
_NS = 16
_NW = _NC * _NS
_K = 5
_EK = N_EDGES // _K
_EW = _EK // _NW
_C = 80
_CH = _EW // _C


def _node_body(x_ref, wlt_ref, bl_ref, w1at_ref, y_ref):
    xl = jnp.dot(x_ref[...], wlt_ref[...], preferred_element_type=jnp.float32)
    xl = xl + bl_ref[...]
    y = jnp.dot(xl, w1at_ref[...], preferred_element_type=jnp.float32)
    yb = y.astype(jnp.bfloat16)
    lo16 = lax.bitcast_convert_type(yb[:, :NH], jnp.uint16)
    hi16 = lax.bitcast_convert_type(yb[:, NH:], jnp.uint16)
    packed = (hi16.astype(jnp.uint32) << 16) | lo16.astype(jnp.uint32)
    y_ref[...] = lax.bitcast_convert_type(packed, jnp.int32)


def _node_precompute(x, WlT, bl2, W1aT):
    return pl.pallas_call(
        _node_body,
        out_shape=jax.ShapeDtypeStruct((N_NODES, NH), jnp.int32),
    )(x, WlT, bl2, W1aT)


_ILV = plsc.PackFormat.INTERLEAVED


def _gather_add_body(y_hbm, src_hbm, dst_hbm, g_hbm,
                     ytab, sidx, didx, sbuf0, dbuf0, sbuf1, dbuf1, obuf,
                     sem_s0, sem_d0, sem_s1, sem_d1):
    cid = lax.axis_index("c")
    sid = lax.axis_index("s")
    wid = sid * _NC + cid

    @pl.when(sid == 0)
    def _():
        pltpu.sync_copy(y_hbm, ytab)

    pltpu.sync_copy(src_hbm.at[wid], sidx)
    pltpu.sync_copy(dst_hbm.at[wid], didx)
    plsc.subcore_barrier()

    def issue(i, sbuf, dbuf, ss, sd):
        pltpu.make_async_copy(ytab.at[sidx.at[i]], sbuf, ss).start()
        pltpu.make_async_copy(ytab.at[didx.at[i]], dbuf, sd).start()

    def drain(sbuf, dbuf, ss, sd):
        pltpu.make_async_copy(y_hbm.at[sidx.at[0]], sbuf, ss).wait()
        pltpu.make_async_copy(y_hbm.at[didx.at[0]], dbuf, sd).wait()

    def compute_store(i, srows, drows):
        def pair(rr, _):
            e0 = rr * 2
            e1 = e0 + 1

            def kchunk(k, _):
                c = k * 16
                s0 = plsc.bitcast(srows[e0, pl.ds(c, 16)], jnp.bfloat16)
                d0 = plsc.bitcast(drows[e0, pl.ds(c, 16)], jnp.bfloat16)
                s1 = plsc.bitcast(srows[e1, pl.ds(c, 16)], jnp.bfloat16)
                d1 = plsc.bitcast(drows[e1, pl.ds(c, 16)], jnp.bfloat16)
                sa0, sb0 = plsc.unpack(s0, format=_ILV)
                da0, db0 = plsc.unpack(d0, format=_ILV)
                sa1, sb1 = plsc.unpack(s1, format=_ILV)
                da1, db1 = plsc.unpack(d1, format=_ILV)
                fa0 = sa0 + da0
                fb0 = sb0 + db0
                fa1 = sa1 + da1
                fb1 = sb1 + db1
                lo_pair = plsc.bitcast(plsc.pack(fa0, fa1, format=_ILV),
                                       jnp.int32)
                hi_pair = plsc.bitcast(plsc.pack(fb0, fb1, format=_ILV),
                                       jnp.int32)
                obuf[rr, pl.ds(c, 16)] = lo_pair
                obuf[rr, pl.ds(NH + c, 16)] = hi_pair
                return 0

            return lax.fori_loop(0, NH // 16, kchunk, 0)

        lax.fori_loop(0, _C // 2, pair, 0)
        base2 = wid * (_EW // 2) + i * (_C // 2)
        pltpu.sync_copy(obuf, g_hbm.at[pl.ds(base2, _C // 2)])

    issue(0, sbuf0, dbuf0, sem_s0, sem_d0)

    def outer(i, _):
        nxt = jnp.minimum(i + 1, _CH - 1)

        @pl.when(i % 2 == 0)
        def _():
            issue(nxt, sbuf1, dbuf1, sem_s1, sem_d1)
            drain(sbuf0, dbuf0, sem_s0, sem_d0)
            compute_store(i, sbuf0, dbuf0)

        @pl.when(i % 2 == 1)
        def _():
            issue(nxt, sbuf0, dbuf0, sem_s0, sem_d0)
            drain(sbuf1, dbuf1, sem_s1, sem_d1)
            compute_store(i, sbuf1, dbuf1)

        return 0

    lax.fori_loop(0, _CH, outer, 0)
    drain(sbuf1, dbuf1, sem_s1, sem_d1)


def _gather_add(y, src3, dst3):
    mesh = plsc.VectorSubcoreMesh(core_axis_name="c", subcore_axis_name="s")
    fn = functools.partial(
        pl.kernel, mesh=mesh,
        compiler_params=pltpu.CompilerParams(
            needs_layout_passes=False, use_tc_tiling_on_sc=False),
        out_type=jax.ShapeDtypeStruct((_EK // 2, NIN), jnp.int32),
        scratch_types=[
            pltpu.VMEM_SHARED((N_NODES, NH), jnp.int32),
            pltpu.VMEM((_CH, _C), jnp.int32),
            pltpu.VMEM((_CH, _C), jnp.int32),
            pltpu.VMEM((_C, NH), jnp.int32),
            pltpu.VMEM((_C, NH), jnp.int32),
            pltpu.VMEM((_C, NH), jnp.int32),
            pltpu.VMEM((_C, NH), jnp.int32),
            pltpu.VMEM((_C // 2, NIN), jnp.int32),
            pltpu.SemaphoreType.DMA,
            pltpu.SemaphoreType.DMA,
            pltpu.SemaphoreType.DMA,
            pltpu.SemaphoreType.DMA,
        ],
    )(_gather_add_body)
    return fn(y, src3, dst3)


_EB = 12800
_NBK = _EK // _EB


def _edge_body(g_ref, ea_ref, w1et_ref, b1_ref, w2t_ref, b2_ref, out_ref):
    gbf = pltpu.bitcast(g_ref[...], jnp.bfloat16)
    gf = gbf.astype(jnp.float32)
    u = jnp.dot(ea_ref[...], w1et_ref[...],
                preferred_element_type=jnp.float32) + b1_ref[...]
    h = jnp.maximum(u + gf, 0.0)
    o = jnp.dot(h, w2t_ref[...], preferred_element_type=jnp.float32) + b2_ref[...]
    out_ref[...] = jnp.maximum(o, 0.0)


def _edge_body_prev(g_ref, ea_ref, w1et_ref, b1_ref, w2t_ref, b2_ref,
                    prev_ref, out_ref):
    del prev_ref
    _edge_body(g_ref, ea_ref, w1et_ref, b1_ref, w2t_ref, b2_ref, out_ref)


def _edge_mlp_slice(k, g_k, edge_attr, W1eT, b12, W2T, b22, out_prev):
    off = k * _NBK
    in_specs = [
        pl.BlockSpec((_EB // 2, NIN), lambda i: (i, 0)),
        pl.BlockSpec((_EB, NIN), lambda i, o=off: (i + o, 0)),
        pl.BlockSpec((NIN, NOUT), lambda i: (0, 0)),
        pl.BlockSpec((1, NOUT), lambda i: (0, 0)),
        pl.BlockSpec((NOUT, NOUT), lambda i: (0, 0)),
        pl.BlockSpec((1, NOUT), lambda i: (0, 0)),
    ]
    args = [g_k, edge_attr, W1eT, b12, W2T, b22]
    body = _edge_body
    aliases = {}
    if out_prev is not None:
        in_specs.append(pl.BlockSpec(memory_space=pl.MemorySpace.ANY))
        args.append(out_prev)
        body = _edge_body_prev
        aliases = {6: 0}
    return pl.pallas_call(
        body,
        grid=(_NBK,),
        in_specs=in_specs,
        out_specs=pl.BlockSpec((_EB, NOUT), lambda i, o=off: (i + o, 0)),
        out_shape=jax.ShapeDtypeStruct((N_EDGES, NOUT), jnp.float32),
        input_output_aliases=aliases,
    )(*args)


def kernel(x, edge_index, edge_attr, Wl, bl, W1, b1, W2, b2):
    src4 = edge_index[0].astype(jnp.int32).reshape(_K, _NW, _CH, _C)
    dst4 = edge_index[1].astype(jnp.int32).reshape(_K, _NW, _CH, _C)
    WlT = Wl.T
    W1aT = W1[:, :NIN].T
    W1eT = W1[:, NIN:].T
    W2T = W2.T
    b12 = b1.reshape(1, NOUT)
    b22 = b2.reshape(1, NOUT)

    y = _node_precompute(x, WlT, bl.reshape(1, NIN), W1aT)
    gs = [_gather_add(y, src4[k], dst4[k]) for k in range(_K)]
    out = None
    for k in range(_K):
        out = _edge_mlp_slice(k, gs[k], edge_attr, W1eT, b12, W2T, b22, out)
    return out
